# baseline (speedup 1.0000x reference)
import functools

import jax
import jax.numpy as jnp
from jax import lax
from jax.experimental import pallas as pl
from jax.experimental.pallas import tpu as pltpu
from jax.experimental.pallas import tpu_sc as plsc

BN = 1024
KC = 8192


def _argmin_body(zm2_ref, zsq_ref, c_ref, csq_ref, iota_ref, w_ref, idx_ref,
                 loss_ref, *, n_tokens, code_dim):
    n = pl.program_id(0)
    nn = pl.num_programs(0)

    m2 = lax.dot_general(zm2_ref[...], c_ref[...],
                         (((1,), (1,)), ((), ())),
                         preferred_element_type=jnp.float32)
    d2 = (zsq_ref[...] + m2) + csq_ref[...]

    minv = jnp.min(d2, axis=1, keepdims=True)
    maskb = (d2 == minv).astype(jnp.bfloat16)
    agg = lax.dot_general(maskb, w_ref[...],
                          (((1,), (0,)), ((), ())),
                          preferred_element_type=jnp.float32)
    idxf = agg[:, 0:1] * 64.0 + agg[:, 1:2]
    cnt = agg[:, 2:3]
    tie = jnp.max(cnt) > 1.5

    @pl.when(jnp.logical_not(tie))
    def _():
        idx_ref[...] = idxf.astype(jnp.int32)

    @pl.when(tie)
    def _():
        big = jnp.float32(1e9)
        idx_ref[...] = jnp.min(jnp.where(d2 == minv, iota_ref[...], big),
                               axis=1, keepdims=True).astype(jnp.int32)

    s = jnp.sum(minv)
    prev = jnp.where(n == 0, jnp.float32(0.0), loss_ref[0, 0])
    tot = prev + s
    loss_ref[0, 0] = jnp.where(n == nn - 1,
                               tot / jnp.float32(n_tokens * code_dim),
                               tot)


def _argmin_call(zm2, z_sq, codebook, c_sq_row, iota_row, w):
    n_tokens, code_dim = zm2.shape
    num_codes = codebook.shape[0]
    assert KC == num_codes
    grid = (n_tokens // BN,)
    body = functools.partial(_argmin_body, n_tokens=n_tokens,
                             code_dim=code_dim)
    return pl.pallas_call(
        body,
        grid=grid,
        in_specs=[
            pl.BlockSpec((BN, code_dim), lambda n: (n, 0)),
            pl.BlockSpec((BN, 1), lambda n: (n, 0)),
            pl.BlockSpec((KC, code_dim), lambda n: (0, 0)),
            pl.BlockSpec((1, KC), lambda n: (0, 0)),
            pl.BlockSpec((1, KC), lambda n: (0, 0)),
            pl.BlockSpec((KC, 8), lambda n: (0, 0)),
        ],
        out_specs=[
            pl.BlockSpec((BN, 1), lambda n: (n, 0)),
            pl.BlockSpec(memory_space=pltpu.SMEM),
        ],
        out_shape=[
            jax.ShapeDtypeStruct((n_tokens, 1), jnp.int32),
            jax.ShapeDtypeStruct((1, 1), jnp.float32),
        ],
        compiler_params=pltpu.CompilerParams(
            dimension_semantics=("arbitrary",)),
    )(zm2, z_sq, codebook, c_sq_row, iota_row, w)


def _make_gather(num_pairs, n_tokens):
    info = plsc.get_sparse_core_info()
    nc, ns = info.num_cores, info.num_subcores
    nw = nc * ns
    rows_per_w = n_tokens // nw
    chunk = 128
    j_per_w = rows_per_w // chunk
    half = j_per_w // 2
    mesh = plsc.VectorSubcoreMesh(core_axis_name="c", subcore_axis_name="s")

    @functools.partial(
        pl.kernel, mesh=mesh,
        out_type=jax.ShapeDtypeStruct((n_tokens, 128), jnp.float32),
        scratch_types=[
            pltpu.VMEM((j_per_w, chunk), jnp.int32),
            pltpu.VMEM((half * chunk, 128), jnp.float32),
            pltpu.SemaphoreType.DMA,
        ],
    )
    def gather(table_hbm, idx_hbm, out_hbm, idx_v, rows_v, sem):
        wid = lax.axis_index("s") * nc + lax.axis_index("c")
        pltpu.sync_copy(idx_hbm.at[pl.ds(wid * j_per_w, j_per_w)], idx_v)
        for h in range(2):
            copies = [
                pltpu.async_copy(table_hbm.at[idx_v.at[h * half + j]],
                                 rows_v.at[pl.ds(j * chunk, chunk)], sem)
                for j in range(half)
            ]
            for c in copies:
                c.wait()
            pltpu.sync_copy(
                rows_v,
                out_hbm.at[pl.ds(wid * rows_per_w + h * half * chunk,
                                 half * chunk)])

    return gather


def kernel(z, codebook):
    n_tokens, code_dim = z.shape
    num_codes = codebook.shape[0]
    z_sq = jnp.sum(z * z, axis=1, keepdims=True)
    c_sq = jnp.sum(codebook * codebook, axis=1)
    iota = jnp.arange(num_codes, dtype=jnp.float32)
    w = jnp.stack([jnp.floor(iota / 64.0), jnp.mod(iota, 64.0),
                   jnp.ones_like(iota)] + [jnp.zeros_like(iota)] * 5,
                  axis=1).astype(jnp.bfloat16)
    idx2d, loss = _argmin_call(z * -2.0, z_sq, codebook, c_sq[None, :],
                               iota[None, :], w)
    indices = idx2d.reshape(-1)
    pair_table = codebook.reshape(-1, 2 * code_dim)
    gather = _make_gather(pair_table.shape[0], n_tokens)
    pairs = gather(pair_table, (indices >> 1).reshape(-1, 128))
    quantized = jnp.where((indices & 1)[:, None] == 1,
                          pairs[:, code_dim:], pairs[:, :code_dim])
    return (quantized, indices, loss[0, 0])


# speedup vs baseline: 1.8888x; 1.8888x over previous
import functools

import jax
import jax.numpy as jnp
from jax import lax
from jax.experimental import pallas as pl
from jax.experimental.pallas import tpu as pltpu
from jax.experimental.pallas import tpu_sc as plsc

BN = 1024
KC = 8192


def _argmin_body(zm2_ref, zsq_ref, c_ref, csq_ref, iota_ref, idx_ref,
                 loss_ref, runmin_ref, runarg_ref, *, n_tokens, code_dim):
    k = pl.program_id(1)
    nk = pl.num_programs(1)
    n = pl.program_id(0)
    nn = pl.num_programs(0)

    m2 = lax.dot_general(zm2_ref[...], c_ref[...],
                         (((1,), (1,)), ((), ())),
                         preferred_element_type=jnp.float32)
    d2 = (zsq_ref[...] + m2) + csq_ref[...]

    minv = jnp.min(d2, axis=1, keepdims=True)
    big = jnp.float32(1e9)
    idxc = jnp.min(jnp.where(d2 == minv, iota_ref[...], big),
                   axis=1, keepdims=True)

    @pl.when(k == 0)
    def _():
        runmin_ref[...] = minv
        runarg_ref[...] = idxc

    @pl.when(k > 0)
    def _():
        better = minv < runmin_ref[...]
        runmin_ref[...] = jnp.where(better, minv, runmin_ref[...])
        runarg_ref[...] = jnp.where(better, idxc, runarg_ref[...])

    @pl.when(k == nk - 1)
    def _():
        idx_ref[...] = runarg_ref[...].astype(jnp.int32)
        s = jnp.sum(runmin_ref[...])
        prev = jnp.where(n == 0, jnp.float32(0.0), loss_ref[0, 0])
        tot = prev + s
        loss_ref[0, 0] = jnp.where(n == nn - 1,
                                   tot / jnp.float32(n_tokens * code_dim),
                                   tot)


def _argmin_call(zm2, z_sq, codebook, c_sq_row, iota_row):
    n_tokens, code_dim = zm2.shape
    num_codes = codebook.shape[0]
    grid = (n_tokens // BN, num_codes // KC)
    body = functools.partial(_argmin_body, n_tokens=n_tokens,
                             code_dim=code_dim)
    return pl.pallas_call(
        body,
        grid=grid,
        in_specs=[
            pl.BlockSpec((BN, code_dim), lambda n, k: (n, 0)),
            pl.BlockSpec((BN, 1), lambda n, k: (n, 0)),
            pl.BlockSpec((KC, code_dim), lambda n, k: (k, 0)),
            pl.BlockSpec((1, KC), lambda n, k: (0, k)),
            pl.BlockSpec((1, KC), lambda n, k: (0, k)),
        ],
        out_specs=[
            pl.BlockSpec((BN, 1), lambda n, k: (n, 0)),
            pl.BlockSpec(memory_space=pltpu.SMEM),
        ],
        out_shape=[
            jax.ShapeDtypeStruct((n_tokens, 1), jnp.int32),
            jax.ShapeDtypeStruct((1, 1), jnp.float32),
        ],
        scratch_shapes=[
            pltpu.VMEM((BN, 1), jnp.float32),
            pltpu.VMEM((BN, 1), jnp.float32),
        ],
        compiler_params=pltpu.CompilerParams(
            dimension_semantics=("arbitrary", "arbitrary")),
    )(zm2, z_sq, codebook, c_sq_row, iota_row)


def _make_gather(num_pairs, n_tokens):
    info = plsc.get_sparse_core_info()
    nc, ns = info.num_cores, info.num_subcores
    nw = nc * ns
    rows_per_w = n_tokens // nw
    chunk = 128
    j_per_w = rows_per_w // chunk
    half = j_per_w // 2
    mesh = plsc.VectorSubcoreMesh(core_axis_name="c", subcore_axis_name="s")

    @functools.partial(
        pl.kernel, mesh=mesh,
        out_type=jax.ShapeDtypeStruct((n_tokens, 128), jnp.float32),
        scratch_types=[
            pltpu.VMEM((j_per_w, chunk), jnp.int32),
            pltpu.VMEM((half * chunk, 128), jnp.float32),
            pltpu.SemaphoreType.DMA,
        ],
    )
    def gather(table_hbm, idx_hbm, out_hbm, idx_v, rows_v, sem):
        wid = lax.axis_index("s") * nc + lax.axis_index("c")
        pltpu.sync_copy(idx_hbm.at[pl.ds(wid * j_per_w, j_per_w)], idx_v)
        for h in range(2):
            copies = [
                pltpu.async_copy(table_hbm.at[idx_v.at[h * half + j]],
                                 rows_v.at[pl.ds(j * chunk, chunk)], sem)
                for j in range(half)
            ]
            for c in copies:
                c.wait()
            pltpu.sync_copy(
                rows_v,
                out_hbm.at[pl.ds(wid * rows_per_w + h * half * chunk,
                                 half * chunk)])

    return gather


def kernel(z, codebook):
    n_tokens, code_dim = z.shape
    num_codes = codebook.shape[0]
    z_sq = jnp.sum(z * z, axis=1, keepdims=True)
    c_sq = jnp.sum(codebook * codebook, axis=1)
    iota_row = jnp.arange(num_codes, dtype=jnp.float32)[None, :]
    idx2d, loss = _argmin_call(z * -2.0, z_sq, codebook, c_sq[None, :],
                               iota_row)
    indices = idx2d.reshape(-1)
    table = jnp.concatenate(
        [codebook, jnp.roll(codebook, -1, axis=0)], axis=1)
    gather = _make_gather(table.shape[0], n_tokens)
    rows = gather(table, idx2d.reshape(-1, 128))
    quantized = rows[:, :code_dim]
    return (quantized, indices, loss[0, 0])
